# Initial kernel scaffold; baseline (speedup 1.0000x reference)
import jax
import jax.numpy as jnp
from jax.experimental import pallas as pl


def kernel(x, tau):
    raise NotImplementedError("write your pallas kernel here")


# speedup vs baseline: 34.5815x; 34.5815x over previous
import functools

import jax
import jax.numpy as jnp
from jax import lax
from jax.experimental import pallas as pl

_TOPK = 0.25


def _topk_mask_kernel(x_ref, o_ref, *, k):
    x = x_ref[...]
    bits = lax.bitcast_convert_type(jnp.abs(x), jnp.int32)
    lo0 = jnp.zeros((1, 1) + bits.shape[2:], jnp.int32)
    hi0 = jnp.full((1, 1) + bits.shape[2:], jnp.int32(0x7FFFFFFF))

    def body(i, c):
        lo, hi = c
        mid = lo + ((hi - lo) >> 1)
        cnt = jnp.sum((bits >= mid).astype(jnp.int32), axis=1, keepdims=True)
        ge = cnt >= k
        return jnp.where(ge, mid, lo), jnp.where(ge, hi, mid)

    lo, _ = lax.fori_loop(0, 31, body, (lo0, hi0))
    o_ref[...] = jnp.where(bits >= lo, x, jnp.zeros_like(x))


def kernel(x, tau):
    n, c, h, w = x.shape
    k = max(int(_TOPK * c), 1)
    p = h * w
    xr = x.reshape(n, c, p)
    sparse = pl.pallas_call(
        functools.partial(_topk_mask_kernel, k=k),
        out_shape=jax.ShapeDtypeStruct((n, c, p), x.dtype),
        grid=(n,),
        in_specs=[pl.BlockSpec((1, c, p), lambda i: (i, 0, 0))],
        out_specs=pl.BlockSpec((1, c, p), lambda i: (i, 0, 0)),
    )(xr).reshape(n, c, h, w)
    tau_arr = jnp.asarray(tau)
    tau_f = tau_arr.astype(x.dtype)
    blended = sparse * tau_f + x * (1.0 - tau_f)
    return jnp.where(tau_arr == 1, sparse, blended)
